# Initial kernel scaffold; baseline (speedup 1.0000x reference)
import jax
import jax.numpy as jnp
from jax.experimental import pallas as pl


def kernel(x):
    raise NotImplementedError("write your pallas kernel here")


# speedup vs baseline: 1.0211x; 1.0211x over previous
import functools

import jax
import jax.numpy as jnp
from jax import lax
from jax.experimental import pallas as pl
from jax.experimental.pallas import tpu as pltpu
from jax.experimental.pallas import tpu_sc as plsc

R = 64
C = 32768
L = 16
NC = 2
NS = 16
NW = NC * NS
ROWS_PER_W = R // NW
UNROLL = 8
NVEC = C // L

_mesh = plsc.VectorSubcoreMesh(core_axis_name="c", subcore_axis_name="s")


def _scan_row(buf):
    lane = lax.iota(jnp.int32, L)
    neg_inf = jnp.full((L,), -jnp.inf, dtype=jnp.float32)

    def body(b, carry):
        mv, mi = carry
        base = b * (UNROLL * L)
        for u in range(UNROLL):
            off = base + u * L
            v = buf[pl.ds(off, L)]
            idx = lane + lax.broadcast(off, (L,))
            gt = v > mv
            mv = jnp.where(gt, v, mv)
            mi = jnp.where(gt, idx, mi)
        return mv, mi

    mv, mi = lax.fori_loop(0, NVEC // UNROLL, body,
                           (neg_inf, jnp.zeros((L,), jnp.int32)))
    for s in (8, 4, 2, 1):
        perm = lane ^ s
        mvp = mv.at[perm].get(mode="promise_in_bounds")
        mip = mi.at[perm].get(mode="promise_in_bounds")
        take = (mvp > mv) | ((mvp == mv) & (mip < mi))
        mv = jnp.where(take, mvp, mv)
        mi = jnp.where(take, mip, mi)
    return mv, mi


@functools.partial(
    pl.kernel,
    mesh=_mesh,
    out_type=[
        jax.ShapeDtypeStruct((NW, L), jnp.float32),
        jax.ShapeDtypeStruct((NW, L), jnp.int32),
    ],
    scratch_types=[
        pltpu.VMEM((C,), jnp.float32),
        pltpu.VMEM((C,), jnp.float32),
        pltpu.VMEM((L,), jnp.float32),
        pltpu.VMEM((L,), jnp.int32),
        pltpu.SemaphoreType.DMA,
        pltpu.SemaphoreType.DMA,
    ],
)
def _topk1_sc(x_hbm, vals_hbm, idxs_hbm, buf0, buf1, vstage, istage,
              sem0, sem1):
    wid = lax.axis_index("s") * NC + lax.axis_index("c")
    row0 = wid * ROWS_PER_W
    cp0 = pltpu.async_copy(x_hbm.at[row0], buf0, sem0)
    cp1 = pltpu.async_copy(x_hbm.at[row0 + 1], buf1, sem1)

    cp0.wait()
    m0, i0 = _scan_row(buf0)
    cp1.wait()
    m1, i1 = _scan_row(buf1)

    lane = lax.iota(jnp.int32, L)
    vvec = jnp.where(lane == 0, m0, m1)
    ivec = jnp.where(lane == 0, i0, i1)
    vstage[...] = vvec
    istage[...] = ivec
    pltpu.sync_copy(vstage, vals_hbm.at[wid])
    pltpu.sync_copy(istage, idxs_hbm.at[wid])


def kernel(x):
    vals_pad, idxs_pad = _topk1_sc(x)
    values = vals_pad[:, :ROWS_PER_W].reshape(R, 1)
    indices = idxs_pad[:, :ROWS_PER_W].reshape(R, 1)
    return values, indices


# speedup vs baseline: 1.1110x; 1.0881x over previous
import functools

import jax
import jax.numpy as jnp
from jax import lax
from jax.experimental import pallas as pl
from jax.experimental.pallas import tpu as pltpu
from jax.experimental.pallas import tpu_sc as plsc

R = 64
C = 32768
L = 16
NC = 2
NS = 16
NW = NC * NS
ROWS_PER_W = R // NW
UNROLL = 8
NVEC = C // L

_mesh = plsc.VectorSubcoreMesh(core_axis_name="c", subcore_axis_name="s")


def _scan_row(buf):
    lane = lax.iota(jnp.int32, L)
    neg_inf = jnp.full((L,), -jnp.inf, dtype=jnp.float32)
    zero = jnp.zeros((L,), jnp.int32)

    def body(b, carry):
        mvs = list(carry[0])
        mbs = list(carry[1])
        bb = lax.broadcast(b, (L,))
        base = b * (UNROLL * L)
        for u in range(UNROLL):
            v = buf[pl.ds(base + u * L, L)]
            gt = v > mvs[u]
            mvs[u] = jnp.maximum(v, mvs[u])
            mbs[u] = jnp.where(gt, bb, mbs[u])
        return tuple(mvs), tuple(mbs)

    mvs, mbs = lax.fori_loop(0, NVEC // UNROLL, body,
                             ((neg_inf,) * UNROLL, (zero,) * UNROLL))

    mv = mvs[0]
    mi = mbs[0] * (UNROLL * L) + lane
    for u in range(1, UNROLL):
        idx_u = mbs[u] * (UNROLL * L) + (lane + u * L)
        better = (mvs[u] > mv) | ((mvs[u] == mv) & (idx_u < mi))
        mv = jnp.where(better, mvs[u], mv)
        mi = jnp.where(better, idx_u, mi)
    for s in (8, 4, 2, 1):
        perm = lane ^ s
        mvp = mv.at[perm].get(mode="promise_in_bounds")
        mip = mi.at[perm].get(mode="promise_in_bounds")
        take = (mvp > mv) | ((mvp == mv) & (mip < mi))
        mv = jnp.where(take, mvp, mv)
        mi = jnp.where(take, mip, mi)
    return mv, mi


@functools.partial(
    pl.kernel,
    mesh=_mesh,
    out_type=[
        jax.ShapeDtypeStruct((NW, L), jnp.float32),
        jax.ShapeDtypeStruct((NW, L), jnp.int32),
    ],
    scratch_types=[
        pltpu.VMEM((C,), jnp.float32),
        pltpu.VMEM((C,), jnp.float32),
        pltpu.VMEM((L,), jnp.float32),
        pltpu.VMEM((L,), jnp.int32),
        pltpu.SemaphoreType.DMA,
        pltpu.SemaphoreType.DMA,
    ],
)
def _topk1_sc(x_hbm, vals_hbm, idxs_hbm, buf0, buf1, vstage, istage,
              sem0, sem1):
    wid = lax.axis_index("s") * NC + lax.axis_index("c")
    row0 = wid * ROWS_PER_W
    cp0 = pltpu.async_copy(x_hbm.at[row0], buf0, sem0)
    cp1 = pltpu.async_copy(x_hbm.at[row0 + 1], buf1, sem1)

    cp0.wait()
    m0, i0 = _scan_row(buf0)
    cp1.wait()
    m1, i1 = _scan_row(buf1)

    lane = lax.iota(jnp.int32, L)
    vvec = jnp.where(lane == 0, m0, m1)
    ivec = jnp.where(lane == 0, i0, i1)
    vstage[...] = vvec
    istage[...] = ivec
    pltpu.sync_copy(vstage, vals_hbm.at[wid])
    pltpu.sync_copy(istage, idxs_hbm.at[wid])


def kernel(x):
    vals_pad, idxs_pad = _topk1_sc(x)
    values = vals_pad[:, :ROWS_PER_W].reshape(R, 1)
    indices = idxs_pad[:, :ROWS_PER_W].reshape(R, 1)
    return values, indices
